# baseline (speedup 1.0000x reference)
import functools

import jax
import jax.numpy as jnp
from jax import lax
from jax.experimental import pallas as pl
from jax.experimental.pallas import tpu as pltpu
from jax.experimental.pallas import tpu_sc as plsc

_N = 10000
_K = 27
_HID = 128
_CIN = 256
_E = 160000
_NK = _N * _K
_KH = _K * _HID

_BLK = 128
_NSUB = 16
_EP = 163840
_NBLK = _EP // _BLK
_NBUF = 4
_BN = 400
_BD = 1152

_mesh = plsc.VectorSubcoreMesh(core_axis_name="c", subcore_axis_name="s")


def _make_sc_pass(add_core_offset: bool, split_blocks: bool, sb: int):
    nbt = (_NBLK // 2 // _NSUB) if split_blocks else (_NBLK // _NSUB)
    nsb = nbt // sb
    assert nsb % 2 == 0

    def body(edg_hbm, tab_hbm, out_hbm,
             edg0, edg1, idx0, idx1, rows0, rows1,
             accum, isem0, isem1, gsem0, gsem1):
        c = lax.axis_index("c")
        s = lax.axis_index("s")

        if split_blocks:
            blk0 = c * (_NBLK // 2) + s * nbt
        else:
            blk0 = s * nbt

        pltpu.async_copy(edg_hbm.at[pl.ds(blk0, sb)], edg0, isem0)

        def _zrow(i, _):
            r = i // 8
            col = (i % 8) * 16
            rows0[r, pl.ds(col, 16)] = jnp.zeros((16,), jnp.float32)
            return 0
        lax.fori_loop(0, _BLK * 8, _zrow, 0)

        @pl.when(s < 10)
        def _init():
            base = s * 1000
            for j in range(7):
                pltpu.sync_copy(rows0, accum.at[pl.ds(base + j * 128, 128)])
            pltpu.sync_copy(rows0.at[pl.ds(0, 104)],
                            accum.at[pl.ds(base + 896, 104)])

        plsc.subcore_barrier()

        rows = (rows0, rows1)
        gsems = (gsem0, gsem1)

        def _superblock(q, eb, isem, oeb, oisem):
            pltpu.make_async_copy(edg_hbm.at[pl.ds(blk0 + q * sb, sb)],
                                  eb, isem).wait()
            @pl.when(q < nsb - 1)
            def _():
                pltpu.async_copy(edg_hbm.at[pl.ds(blk0 + (q + 1) * sb, sb)],
                                 oeb, oisem)
            ib = idx0 if eb is edg0 else idx1
            def _mkidx(b, _):
                for j in range(_BLK // 16):
                    sl = pl.ds(j * 16, 16)
                    gi = eb[b, 0, sl] * _K + eb[b, 1, sl]
                    if add_core_offset:
                        gi = gi + c * _NK
                    ib[b, sl] = gi
                return 0
            lax.fori_loop(0, sb, _mkidx, 0)
            pltpu.async_copy(tab_hbm.at[ib.at[0]], rows[0], gsems[0])
            for k in range(sb):
                if k + 1 < sb:
                    pltpu.async_copy(tab_hbm.at[ib.at[k + 1]],
                                     rows[(k + 1) % 2], gsems[(k + 1) % 2])
                pltpu.make_async_copy(tab_hbm.at[ib.at[k]],
                                      rows[k % 2], gsems[k % 2]).wait()
                pltpu.sync_copy(rows[k % 2], accum.at[eb.at[k, 2]], add=True)

        def _pair(p, _):
            _superblock(2 * p, edg0, isem0, edg1, isem1)
            _superblock(2 * p + 1, edg1, isem1, edg0, isem0)
            return 0
        lax.fori_loop(0, nsb // 2, _pair, 0)

        plsc.subcore_barrier()

        @pl.when(s < 10)
        def _flush():
            base = s * 1000
            pltpu.sync_copy(accum.at[pl.ds(base, 1000)],
                            out_hbm.at[pl.ds(c * _N + base, 1000)])

    return pl.kernel(
        body,
        out_type=jax.ShapeDtypeStruct((2 * _N, _HID), jnp.float32),
        mesh=_mesh,
        scratch_types=[
            pltpu.VMEM((sb, 3, _BLK), jnp.int32),
            pltpu.VMEM((sb, 3, _BLK), jnp.int32),
            pltpu.VMEM((sb, _BLK), jnp.int32),
            pltpu.VMEM((sb, _BLK), jnp.int32),
            pltpu.VMEM((_BLK, _HID), jnp.float32),
            pltpu.VMEM((_BLK, _HID), jnp.float32),
            pltpu.VMEM_SHARED((_N + 8, _HID), jnp.float32),
            pltpu.SemaphoreType.DMA,
            pltpu.SemaphoreType.DMA,
            pltpu.SemaphoreType.DMA,
            pltpu.SemaphoreType.DMA,
        ],
    )


_sc_pass_zr = _make_sc_pass(add_core_offset=True, split_blocks=False, sb=8)
_sc_pass_q = _make_sc_pass(add_core_offset=False, split_blocks=True, sb=4)


def _zr_mm_body(hx_ref, w_ref, out_ref):
    out_ref[0] = jnp.dot(hx_ref[...], w_ref[0],
                         preferred_element_type=jnp.float32)


_zr_mm = pl.pallas_call(
    _zr_mm_body,
    grid=(2, _KH // _BD, _N // _BN),
    in_specs=[
        pl.BlockSpec((_BN, _CIN), lambda zr, j, i: (i, 0)),
        pl.BlockSpec((1, _CIN, _BD), lambda zr, j, i: (zr, 0, j)),
    ],
    out_specs=pl.BlockSpec((1, _BN, _BD), lambda zr, j, i: (zr, i, j)),
    out_shape=jax.ShapeDtypeStruct((2, _N, _KH), jnp.float32),
)


def _q_mm_body(pr_ref, h_ref, x_ref, br_ref, wh_ref, wx_ref, out_ref):
    r = jax.nn.sigmoid(pr_ref[...] + br_ref[0])
    rh = r * h_ref[...]
    out_ref[...] = (
        jnp.dot(rh, wh_ref[...], preferred_element_type=jnp.float32)
        + jnp.dot(x_ref[...], wx_ref[...], preferred_element_type=jnp.float32))


_q_mm = pl.pallas_call(
    _q_mm_body,
    grid=(_KH // _BD, _N // _BN),
    in_specs=[
        pl.BlockSpec((_BN, _HID), lambda j, i: (i, 0)),
        pl.BlockSpec((_BN, _HID), lambda j, i: (i, 0)),
        pl.BlockSpec((_BN, _HID), lambda j, i: (i, 0)),
        pl.BlockSpec((1, _HID), lambda j, i: (0, 0)),
        pl.BlockSpec((_HID, _BD), lambda j, i: (0, j)),
        pl.BlockSpec((_HID, _BD), lambda j, i: (0, j)),
    ],
    out_specs=pl.BlockSpec((_BN, _BD), lambda j, i: (i, j)),
    out_shape=jax.ShapeDtypeStruct((_N, _KH), jnp.float32),
)


def _gate_body(pz_ref, q0_ref, q1_ref, h_ref, bz_ref, bq_ref, out_ref):
    z = jax.nn.sigmoid(pz_ref[...] + bz_ref[0])
    q = jnp.tanh(q0_ref[...] + q1_ref[...] + bq_ref[0])
    out_ref[...] = (1.0 - z) * h_ref[...] + z * q


_gate = pl.pallas_call(
    _gate_body,
    grid=(_N // _BN,),
    in_specs=[
        pl.BlockSpec((_BN, _HID), lambda i: (i, 0)),
        pl.BlockSpec((_BN, _HID), lambda i: (i, 0)),
        pl.BlockSpec((_BN, _HID), lambda i: (i, 0)),
        pl.BlockSpec((_BN, _HID), lambda i: (i, 0)),
        pl.BlockSpec((1, _HID), lambda i: (0, 0)),
        pl.BlockSpec((1, _HID), lambda i: (0, 0)),
    ],
    out_specs=pl.BlockSpec((_BN, _HID), lambda i: (i, 0)),
    out_shape=jax.ShapeDtypeStruct((_N, _HID), jnp.float32),
)


def kernel(h, x, edge_index, edge_kernel, Wz, bz, Wr, br, Wq, bq):
    hx = jnp.concatenate([h, x], axis=1)
    wzf = Wz.transpose(1, 0, 2).reshape(_CIN, _KH)
    wrf = Wr.transpose(1, 0, 2).reshape(_CIN, _KH)
    wzr = jnp.stack([wzf, wrf])
    wqh = Wq[:, :_HID, :].transpose(1, 0, 2).reshape(_HID, _KH)
    wqx = Wq[:, _HID:, :].transpose(1, 0, 2).reshape(_HID, _KH)

    pad = _EP - _E
    zpad = jnp.zeros((pad,), jnp.int32)
    src = jnp.concatenate([edge_index[0], zpad]).reshape(_NBLK, 1, _BLK)
    dst = jnp.concatenate([edge_index[1], jnp.full((pad,), _N, jnp.int32)]
                          ).reshape(_NBLK, 1, _BLK)
    kern = jnp.concatenate([edge_kernel, zpad]).reshape(_NBLK, 1, _BLK)
    edg = jnp.concatenate([src, kern, dst], axis=1)

    t1 = _zr_mm(hx, wzr).reshape(2 * _NK, _HID)
    pre = _sc_pass_zr(edg, t1)
    pz, pr = pre[:_N], pre[_N:]
    t2 = _q_mm(pr, h, x, br.reshape(1, _HID), wqh, wqx).reshape(_NK, _HID)
    qp = _sc_pass_q(edg, t2)
    return _gate(pz, qp[:_N], qp[_N:], h,
                 bz.reshape(1, _HID), bq.reshape(1, _HID))


# speedup vs baseline: 1.0867x; 1.0867x over previous
import functools

import jax
import jax.numpy as jnp
from jax import lax
from jax.experimental import pallas as pl
from jax.experimental.pallas import tpu as pltpu
from jax.experimental.pallas import tpu_sc as plsc

_N = 10000
_K = 27
_HID = 128
_CIN = 256
_E = 160000
_NK = _N * _K
_KH = _K * _HID

_BLK = 128
_NSUB = 16
_EP = 163840
_NBLK = _EP // _BLK
_NBUF = 4
_BN = 400
_BD = 1152

_mesh = plsc.VectorSubcoreMesh(core_axis_name="c", subcore_axis_name="s")


def _make_sc_pass(add_core_offset: bool, split_blocks: bool, sb: int):
    nbt = (_NBLK // 2 // _NSUB) if split_blocks else (_NBLK // _NSUB)
    nsb = nbt // sb
    assert nsb % 2 == 0

    def body(edg_hbm, tab_hbm, out_hbm,
             edg0, edg1, idx0, idx1, rows0, rows1,
             accum, isem0, isem1, gsem0, gsem1):
        c = lax.axis_index("c")
        s = lax.axis_index("s")

        if split_blocks:
            blk0 = c * (_NBLK // 2) + s * nbt
        else:
            blk0 = s * nbt

        pltpu.async_copy(edg_hbm.at[pl.ds(blk0, sb)], edg0, isem0)

        def _zrow(i, _):
            r = i // 8
            col = (i % 8) * 16
            rows0[r, pl.ds(col, 16)] = jnp.zeros((16,), jnp.float32)
            return 0
        lax.fori_loop(0, _BLK * 8, _zrow, 0)

        @pl.when(s < 10)
        def _init():
            base = s * 1000
            for j in range(7):
                pltpu.sync_copy(rows0, accum.at[pl.ds(base + j * 128, 128)])
            pltpu.sync_copy(rows0.at[pl.ds(0, 104)],
                            accum.at[pl.ds(base + 896, 104)])

        plsc.subcore_barrier()

        rows = (rows0, rows1)
        gsems = (gsem0, gsem1)

        def _superblock(q, eb, isem, oeb, oisem):
            pltpu.make_async_copy(edg_hbm.at[pl.ds(blk0 + q * sb, sb)],
                                  eb, isem).wait()
            @pl.when(q < nsb - 1)
            def _():
                pltpu.async_copy(edg_hbm.at[pl.ds(blk0 + (q + 1) * sb, sb)],
                                 oeb, oisem)
            ib = idx0 if eb is edg0 else idx1
            def _mkidx(b, _):
                for j in range(_BLK // 16):
                    sl = pl.ds(j * 16, 16)
                    gi = eb[b, 0, sl] * _K + eb[b, 1, sl]
                    if add_core_offset:
                        gi = gi + c * _NK
                    ib[b, sl] = gi
                return 0
            lax.fori_loop(0, sb, _mkidx, 0)
            pltpu.async_copy(tab_hbm.at[ib.at[0]], rows[0], gsems[0])
            for k in range(sb):
                if k + 1 < sb:
                    pltpu.async_copy(tab_hbm.at[ib.at[k + 1]],
                                     rows[(k + 1) % 2], gsems[(k + 1) % 2])
                pltpu.make_async_copy(tab_hbm.at[ib.at[k]],
                                      rows[k % 2], gsems[k % 2]).wait()
                pltpu.sync_copy(rows[k % 2], accum.at[eb.at[k, 2]], add=True)

        def _pair(p, _):
            _superblock(2 * p, edg0, isem0, edg1, isem1)
            _superblock(2 * p + 1, edg1, isem1, edg0, isem0)
            return 0
        lax.fori_loop(0, nsb // 2, _pair, 0)

        plsc.subcore_barrier()

        @pl.when(s < 10)
        def _flush():
            base = s * 1000
            pltpu.sync_copy(accum.at[pl.ds(base, 1000)],
                            out_hbm.at[pl.ds(c * _N + base, 1000)])

    return pl.kernel(
        body,
        out_type=jax.ShapeDtypeStruct((2 * _N, _HID), jnp.float32),
        mesh=_mesh,
        scratch_types=[
            pltpu.VMEM((sb, 3, _BLK), jnp.int32),
            pltpu.VMEM((sb, 3, _BLK), jnp.int32),
            pltpu.VMEM((sb, _BLK), jnp.int32),
            pltpu.VMEM((sb, _BLK), jnp.int32),
            pltpu.VMEM((_BLK, _HID), jnp.float32),
            pltpu.VMEM((_BLK, _HID), jnp.float32),
            pltpu.VMEM_SHARED((_N + 8, _HID), jnp.float32),
            pltpu.SemaphoreType.DMA,
            pltpu.SemaphoreType.DMA,
            pltpu.SemaphoreType.DMA,
            pltpu.SemaphoreType.DMA,
        ],
    )


_sc_pass_zr = _make_sc_pass(add_core_offset=True, split_blocks=False, sb=8)
_sc_pass_q = _make_sc_pass(add_core_offset=False, split_blocks=True, sb=4)


def _zr_mm_body(hx_ref, w_ref, out_ref):
    out_ref[0] = jnp.dot(hx_ref[...], w_ref[0],
                         preferred_element_type=jnp.float32)


_zr_mm = pl.pallas_call(
    _zr_mm_body,
    grid=(2, _KH // _BD, _N // _BN),
    in_specs=[
        pl.BlockSpec((_BN, _CIN), lambda zr, j, i: (i, 0)),
        pl.BlockSpec((1, _CIN, _BD), lambda zr, j, i: (zr, 0, j)),
    ],
    out_specs=pl.BlockSpec((1, _BN, _BD), lambda zr, j, i: (zr, i, j)),
    out_shape=jax.ShapeDtypeStruct((2, _N, _KH), jnp.float32),
)


def _q_mm_body(pr_ref, h_ref, x_ref, br_ref, wh_ref, wx_ref, out_ref):
    r = jax.nn.sigmoid(pr_ref[...] + br_ref[0])
    rh = r * h_ref[...]
    out_ref[...] = (
        jnp.dot(rh, wh_ref[...], preferred_element_type=jnp.float32)
        + jnp.dot(x_ref[...], wx_ref[...], preferred_element_type=jnp.float32))


_q_mm = pl.pallas_call(
    _q_mm_body,
    grid=(_KH // _BD, _N // _BN),
    in_specs=[
        pl.BlockSpec((_BN, _HID), lambda j, i: (i, 0)),
        pl.BlockSpec((_BN, _HID), lambda j, i: (i, 0)),
        pl.BlockSpec((_BN, _HID), lambda j, i: (i, 0)),
        pl.BlockSpec((1, _HID), lambda j, i: (0, 0)),
        pl.BlockSpec((_HID, _BD), lambda j, i: (0, j)),
        pl.BlockSpec((_HID, _BD), lambda j, i: (0, j)),
    ],
    out_specs=pl.BlockSpec((_BN, _BD), lambda j, i: (i, j)),
    out_shape=jax.ShapeDtypeStruct((_N, _KH), jnp.float32),
)


def _gate_body(pz_ref, q0_ref, q1_ref, h_ref, bz_ref, bq_ref, out_ref):
    z = jax.nn.sigmoid(pz_ref[...] + bz_ref[0])
    q = jnp.tanh(q0_ref[...] + q1_ref[...] + bq_ref[0])
    out_ref[...] = (1.0 - z) * h_ref[...] + z * q


_gate = pl.pallas_call(
    _gate_body,
    grid=(_N // _BN,),
    in_specs=[
        pl.BlockSpec((_BN, _HID), lambda i: (i, 0)),
        pl.BlockSpec((_BN, _HID), lambda i: (i, 0)),
        pl.BlockSpec((_BN, _HID), lambda i: (i, 0)),
        pl.BlockSpec((_BN, _HID), lambda i: (i, 0)),
        pl.BlockSpec((1, _HID), lambda i: (0, 0)),
        pl.BlockSpec((1, _HID), lambda i: (0, 0)),
    ],
    out_specs=pl.BlockSpec((_BN, _HID), lambda i: (i, 0)),
    out_shape=jax.ShapeDtypeStruct((_N, _HID), jnp.float32),
)


def kernel(h, x, edge_index, edge_kernel, Wz, bz, Wr, br, Wq, bq):
    hx = jnp.concatenate([h, x], axis=1)
    wzf = Wz.transpose(1, 0, 2).reshape(_CIN, _KH)
    wrf = Wr.transpose(1, 0, 2).reshape(_CIN, _KH)
    wzr = jnp.stack([wzf, wrf])
    wqh = Wq[:, :_HID, :].transpose(1, 0, 2).reshape(_HID, _KH)
    wqx = Wq[:, _HID:, :].transpose(1, 0, 2).reshape(_HID, _KH)

    nchunk = 32
    chunk = _E // nchunk
    cpad = (_EP - _E) // nchunk
    src = jnp.pad(edge_index[0].reshape(nchunk, chunk), ((0, 0), (0, cpad)))
    kern = jnp.pad(edge_kernel.reshape(nchunk, chunk), ((0, 0), (0, cpad)))
    dpad = jnp.broadcast_to(_N + (jnp.arange(cpad, dtype=jnp.int32) % 8),
                            (nchunk, cpad))
    dst = jnp.concatenate([edge_index[1].reshape(nchunk, chunk), dpad], axis=1)
    edg = jnp.stack([src.reshape(_NBLK, _BLK), kern.reshape(_NBLK, _BLK),
                     dst.reshape(_NBLK, _BLK)], axis=1)

    t1 = _zr_mm(hx, wzr).reshape(2 * _NK, _HID)
    pre = _sc_pass_zr(edg, t1)
    pz, pr = pre[:_N], pre[_N:]
    t2 = _q_mm(pr, h, x, br.reshape(1, _HID), wqh, wqx).reshape(_NK, _HID)
    qp = _sc_pass_q(edg, t2)
    return _gate(pz, qp[:_N], qp[_N:], h,
                 bz.reshape(1, _HID), bq.reshape(1, _HID))
